# baseline (speedup 1.0000x reference)
import functools
from typing import Any

import jax
import jax.numpy as jnp
import numpy as np
from jax.experimental import pallas as pl
from jax.experimental.pallas import tpu as pltpu

B, S, D, H, E, K, FF = 1, 2048, 1024, 16, 8, 2, 2048
DH = D // H
EPS = 1e-5

BM = 256
BN = 512
MOE_BM = 128
NPAD = (K * S // MOE_BM + E) * MOE_BM
N_MOE_BLOCKS = NPAD // MOE_BM


def _ln_qkv_kernel(x_ref, w_ref, wqkv_ref, o_ref):
    x = x_ref[...]
    var = jnp.mean(x * x, axis=1, keepdims=True)
    xn = x * jax.lax.rsqrt(var + EPS) * w_ref[0, :]
    o_ref[...] = jnp.dot(xn, wqkv_ref[...], preferred_element_type=jnp.float32)


def _ln_qkv(x, ln_w, wqkv):
    return pl.pallas_call(
        _ln_qkv_kernel,
        grid=(S // BM, (3 * D) // BN),
        in_specs=[
            pl.BlockSpec((BM, D), lambda m, n: (m, 0)),
            pl.BlockSpec((1, D), lambda m, n: (0, 0)),
            pl.BlockSpec((D, BN), lambda m, n: (0, n)),
        ],
        out_specs=pl.BlockSpec((BM, BN), lambda m, n: (m, n)),
        out_shape=jax.ShapeDtypeStruct((S, 3 * D), jnp.float32),
    )(x, ln_w.reshape(1, D), wqkv)


def _attn_kernel(q_ref, k_ref, v_ref, o_ref):
    qb = pl.program_id(1)
    q = q_ref[0]
    k = k_ref[0]
    v = v_ref[0]
    s = jax.lax.dot_general(q, k, (((1,), (1,)), ((), ())),
                            preferred_element_type=jnp.float32)
    s = s * (1.0 / np.sqrt(DH))
    row = qb * BM + jax.lax.broadcasted_iota(jnp.int32, (BM, S), 0)
    col = jax.lax.broadcasted_iota(jnp.int32, (BM, S), 1)
    s = jnp.where(col <= row, s, -1e9)
    m = jnp.max(s, axis=1, keepdims=True)
    p = jnp.exp(s - m)
    p = p / jnp.sum(p, axis=1, keepdims=True)
    o_ref[0] = jnp.dot(p, v, preferred_element_type=jnp.float32)


def _attention(q, k, v):
    return pl.pallas_call(
        _attn_kernel,
        grid=(H, S // BM),
        in_specs=[
            pl.BlockSpec((1, BM, DH), lambda h, qb: (h, qb, 0)),
            pl.BlockSpec((1, S, DH), lambda h, qb: (h, 0, 0)),
            pl.BlockSpec((1, S, DH), lambda h, qb: (h, 0, 0)),
        ],
        out_specs=pl.BlockSpec((1, BM, DH), lambda h, qb: (h, qb, 0)),
        out_shape=jax.ShapeDtypeStruct((H, S, DH), jnp.float32),
    )(q, k, v)


def _proj_res_kernel(a_ref, w_ref, r_ref, o_ref):
    o_ref[...] = jnp.dot(a_ref[...], w_ref[...],
                         preferred_element_type=jnp.float32) + r_ref[...]


def _proj_residual(a, w, res):
    return pl.pallas_call(
        _proj_res_kernel,
        grid=(S // BM, D // BN),
        in_specs=[
            pl.BlockSpec((BM, D), lambda m, n: (m, 0)),
            pl.BlockSpec((D, BN), lambda m, n: (0, n)),
            pl.BlockSpec((BM, BN), lambda m, n: (m, n)),
        ],
        out_specs=pl.BlockSpec((BM, BN), lambda m, n: (m, n)),
        out_shape=jax.ShapeDtypeStruct((S, D), jnp.float32),
    )(a, w, res)


def _ln_router_kernel(x_ref, w_ref, wr_ref, xn_ref, lg_ref):
    x = x_ref[...]
    var = jnp.mean(x * x, axis=1, keepdims=True)
    xn = x * jax.lax.rsqrt(var + EPS) * w_ref[0, :]
    xn_ref[...] = xn
    lg_ref[...] = jnp.dot(xn, wr_ref[...], preferred_element_type=jnp.float32)


def _ln_router(x, ln_w, wr_pad):
    return pl.pallas_call(
        _ln_router_kernel,
        grid=(S // BM,),
        in_specs=[
            pl.BlockSpec((BM, D), lambda m: (m, 0)),
            pl.BlockSpec((1, D), lambda m: (0, 0)),
            pl.BlockSpec((D, 128), lambda m: (0, 0)),
        ],
        out_specs=[
            pl.BlockSpec((BM, D), lambda m: (m, 0)),
            pl.BlockSpec((BM, 128), lambda m: (m, 0)),
        ],
        out_shape=[
            jax.ShapeDtypeStruct((S, D), jnp.float32),
            jax.ShapeDtypeStruct((S, 128), jnp.float32),
        ],
    )(x, ln_w.reshape(1, D), wr_pad)


def _moe1_kernel(e_ref, x_ref, w1_ref, w3_ref, g_ref, o_ref):
    del e_ref
    x = x_ref[...]
    h1 = jnp.dot(x, w1_ref[0], preferred_element_type=jnp.float32)
    h3 = jnp.dot(x, w3_ref[0], preferred_element_type=jnp.float32)
    gate = g_ref[0, 0, :]
    o_ref[...] = (h1 * jax.nn.sigmoid(h1)) * h3 * gate[:, None]


def _moe_stage1(block_e, x_pad, w1, w3, gate_r):
    grid_spec = pltpu.PrefetchScalarGridSpec(
        num_scalar_prefetch=1,
        grid=(FF // BN, N_MOE_BLOCKS),
        in_specs=[
            pl.BlockSpec((MOE_BM, D), lambda n, i, e: (i, 0)),
            pl.BlockSpec((1, D, BN), lambda n, i, e: (e[i], 0, n)),
            pl.BlockSpec((1, D, BN), lambda n, i, e: (e[i], 0, n)),
            pl.BlockSpec((1, 1, MOE_BM), lambda n, i, e: (i, 0, 0)),
        ],
        out_specs=pl.BlockSpec((MOE_BM, BN), lambda n, i, e: (i, n)),
    )
    return pl.pallas_call(
        _moe1_kernel,
        grid_spec=grid_spec,
        out_shape=jax.ShapeDtypeStruct((NPAD, FF), jnp.float32),
    )(block_e, x_pad, w1, w3, gate_r)


def _moe2_kernel(e_ref, a_ref, w2_ref, o_ref):
    del e_ref
    o_ref[...] = jnp.dot(a_ref[...], w2_ref[0],
                         preferred_element_type=jnp.float32)


def _moe_stage2(block_e, act, w2):
    grid_spec = pltpu.PrefetchScalarGridSpec(
        num_scalar_prefetch=1,
        grid=(D // BN, N_MOE_BLOCKS),
        in_specs=[
            pl.BlockSpec((MOE_BM, FF), lambda n, i, e: (i, 0)),
            pl.BlockSpec((1, FF, BN), lambda n, i, e: (e[i], 0, n)),
        ],
        out_specs=pl.BlockSpec((MOE_BM, BN), lambda n, i, e: (i, n)),
    )
    return pl.pallas_call(
        _moe2_kernel,
        grid_spec=grid_spec,
        out_shape=jax.ShapeDtypeStruct((NPAD, D), jnp.float32),
    )(block_e, act, w2)


def _rope_tables():
    half = DH // 2
    inv = 1.0 / (10000.0 ** (np.arange(half, dtype=np.float32) / half))
    t = np.arange(S, dtype=np.float32)[:, None] * inv[None, :]
    return jnp.asarray(np.cos(t)), jnp.asarray(np.sin(t))


def _apply_rope(x, cos, sin):
    xh = x.reshape(S, H, DH)
    half = DH // 2
    x1, x2 = xh[..., :half], xh[..., half:]
    c = cos[:, None, :]
    s = sin[:, None, :]
    out = jnp.concatenate([x1 * c - x2 * s, x2 * c + x1 * s], axis=-1)
    return out.reshape(S, D)


def kernel(hidden_states, sequence_mask, load_balancing_loss, ln1_w, ln2_w,
           Wq, Wk, Wv, Wo, Wr, W1, W3, W2):
    x0 = hidden_states.reshape(S, D)

    wqkv = jnp.concatenate([Wq, Wk, Wv], axis=1)
    qkv = _ln_qkv(x0, ln1_w, wqkv)
    cos, sin = _rope_tables()
    q = _apply_rope(qkv[:, :D], cos, sin).reshape(S, H, DH).transpose(1, 0, 2)
    k = _apply_rope(qkv[:, D:2 * D], cos, sin).reshape(S, H, DH).transpose(1, 0, 2)
    v = qkv[:, 2 * D:].reshape(S, H, DH).transpose(1, 0, 2)
    attn = _attention(q, k, v)
    attn = attn.transpose(1, 0, 2).reshape(S, D)
    hidden = _proj_residual(attn, Wo, x0)

    wr_pad = jnp.zeros((D, 128), jnp.float32).at[:, :E].set(Wr)
    x2, logits_pad = _ln_router(hidden, ln2_w, wr_pad)
    logits = logits_pad[:, :E]
    probs = jax.nn.softmax(logits, axis=-1)
    topv, topi = jax.lax.top_k(probs, K)
    topvn = topv / jnp.sum(topv, axis=-1, keepdims=True)

    assign = jax.nn.one_hot(topi, E, dtype=jnp.float32).sum(axis=1)
    f = jnp.mean(assign, axis=0)
    P = jnp.mean(probs, axis=0)
    lbl = E * jnp.sum(f * P) / K

    T = S
    e_flat = topi.reshape(-1)
    gate_flat = topvn.reshape(-1)
    tok_flat = jnp.arange(T * K, dtype=jnp.int32) // K
    order = jnp.argsort(e_flat)
    sorted_e = e_flat[order]
    cnt = jnp.sum(jax.nn.one_hot(e_flat, E, dtype=jnp.int32), axis=0)
    off = jnp.concatenate([jnp.zeros((1,), jnp.int32), jnp.cumsum(cnt)[:-1]])
    padded_cnt = ((cnt + MOE_BM - 1) // MOE_BM) * MOE_BM
    pstart = jnp.concatenate(
        [jnp.zeros((1,), jnp.int32), jnp.cumsum(padded_cnt)[:-1]])
    rank = jnp.arange(T * K, dtype=jnp.int32) - off[sorted_e]
    dest = pstart[sorted_e] + rank
    tok_pad = jnp.zeros((NPAD,), jnp.int32).at[dest].set(tok_flat[order])
    gate_pad = jnp.zeros((NPAD,), jnp.float32).at[dest].set(gate_flat[order])
    inv = jnp.zeros((T * K,), jnp.int32).at[order].set(dest)
    block_e = (jnp.searchsorted(
        pstart, jnp.arange(N_MOE_BLOCKS, dtype=jnp.int32) * MOE_BM,
        side='right') - 1).astype(jnp.int32)

    x_pad = x2[tok_pad]
    gate_r = gate_pad.reshape(N_MOE_BLOCKS, 1, MOE_BM)
    act = _moe_stage1(block_e, x_pad, w1=W1, w3=W3, gate_r=gate_r)
    y_pad = _moe_stage2(block_e, act, W2)

    inv2 = inv.reshape(T, K)
    out = y_pad[inv2[:, 0]] + y_pad[inv2[:, 1]] + hidden

    return (out.reshape(B, S, D), sequence_mask,
            load_balancing_loss + lbl.reshape(load_balancing_loss.shape))


# speedup vs baseline: 1.0453x; 1.0453x over previous
import functools
from typing import Any

import jax
import jax.numpy as jnp
import numpy as np
from jax.experimental import pallas as pl
from jax.experimental.pallas import tpu as pltpu

B, S, D, H, E, K, FF = 1, 2048, 1024, 16, 8, 2, 2048
DH = D // H
EPS = 1e-5

BM = 256
BN = 512
MOE_BM = 128
NPAD = (K * S // MOE_BM + E) * MOE_BM
N_MOE_BLOCKS = NPAD // MOE_BM


def _ln_qkv_kernel(x_ref, w_ref, wqkv_ref, o_ref):
    x = x_ref[...]
    var = jnp.mean(x * x, axis=1, keepdims=True)
    xn = (x * jax.lax.rsqrt(var + EPS) * w_ref[0, :]).astype(jnp.bfloat16)
    o_ref[...] = jnp.dot(xn, wqkv_ref[...], preferred_element_type=jnp.float32)


def _ln_qkv(x, ln_w, wqkv):
    return pl.pallas_call(
        _ln_qkv_kernel,
        grid=(S // BM, (3 * D) // BN),
        in_specs=[
            pl.BlockSpec((BM, D), lambda m, n: (m, 0)),
            pl.BlockSpec((1, D), lambda m, n: (0, 0)),
            pl.BlockSpec((D, BN), lambda m, n: (0, n)),
        ],
        out_specs=pl.BlockSpec((BM, BN), lambda m, n: (m, n)),
        out_shape=jax.ShapeDtypeStruct((S, 3 * D), jnp.float32),
    )(x, ln_w.reshape(1, D), wqkv.astype(jnp.bfloat16))


def _attn_kernel(q_ref, k_ref, v_ref, o_ref):
    qb = pl.program_id(1)
    q = q_ref[0]
    k = k_ref[0]
    v = v_ref[0]
    s = jax.lax.dot_general(q, k, (((1,), (1,)), ((), ())),
                            preferred_element_type=jnp.float32)
    s = s * (1.0 / np.sqrt(DH))
    row = qb * BM + jax.lax.broadcasted_iota(jnp.int32, (BM, S), 0)
    col = jax.lax.broadcasted_iota(jnp.int32, (BM, S), 1)
    s = jnp.where(col <= row, s, -1e9)
    m = jnp.max(s, axis=1, keepdims=True)
    p = jnp.exp(s - m)
    p = p / jnp.sum(p, axis=1, keepdims=True)
    o_ref[0] = jnp.dot(p.astype(jnp.bfloat16), v,
                       preferred_element_type=jnp.float32)


def _attention(q, k, v):
    return pl.pallas_call(
        _attn_kernel,
        grid=(H, S // BM),
        in_specs=[
            pl.BlockSpec((1, BM, DH), lambda h, qb: (h, qb, 0)),
            pl.BlockSpec((1, S, DH), lambda h, qb: (h, 0, 0)),
            pl.BlockSpec((1, S, DH), lambda h, qb: (h, 0, 0)),
        ],
        out_specs=pl.BlockSpec((1, BM, DH), lambda h, qb: (h, qb, 0)),
        out_shape=jax.ShapeDtypeStruct((H, S, DH), jnp.float32),
    )(q, k, v)


def _proj_res_kernel(a_ref, w_ref, r_ref, o_ref):
    o_ref[...] = jnp.dot(a_ref[...], w_ref[...],
                         preferred_element_type=jnp.float32) + r_ref[...]


def _silu(h):
    return h * jax.nn.sigmoid(h)


def _proj_residual(a, w, res):
    return pl.pallas_call(
        _proj_res_kernel,
        grid=(S // BM, D // BN),
        in_specs=[
            pl.BlockSpec((BM, D), lambda m, n: (m, 0)),
            pl.BlockSpec((D, BN), lambda m, n: (0, n)),
            pl.BlockSpec((BM, BN), lambda m, n: (m, n)),
        ],
        out_specs=pl.BlockSpec((BM, BN), lambda m, n: (m, n)),
        out_shape=jax.ShapeDtypeStruct((S, D), jnp.float32),
    )(a, w, res)


def _ln_router_kernel(x_ref, w_ref, wr_ref, xn_ref, lg_ref):
    x = x_ref[...]
    var = jnp.mean(x * x, axis=1, keepdims=True)
    xn = x * jax.lax.rsqrt(var + EPS) * w_ref[0, :]
    xn_ref[...] = xn
    lg_ref[...] = jnp.dot(xn, wr_ref[...], preferred_element_type=jnp.float32)


def _ln_router(x, ln_w, wr_pad):
    return pl.pallas_call(
        _ln_router_kernel,
        grid=(S // BM,),
        in_specs=[
            pl.BlockSpec((BM, D), lambda m: (m, 0)),
            pl.BlockSpec((1, D), lambda m: (0, 0)),
            pl.BlockSpec((D, 128), lambda m: (0, 0)),
        ],
        out_specs=[
            pl.BlockSpec((BM, D), lambda m: (m, 0)),
            pl.BlockSpec((BM, 128), lambda m: (m, 0)),
        ],
        out_shape=[
            jax.ShapeDtypeStruct((S, D), jnp.float32),
            jax.ShapeDtypeStruct((S, 128), jnp.float32),
        ],
    )(x, ln_w.reshape(1, D), wr_pad)


def _moe1_kernel(e_ref, x_ref, w1_ref, w3_ref, g_ref, o_ref):
    del e_ref
    x = x_ref[...]
    h1 = jnp.dot(x, w1_ref[0], preferred_element_type=jnp.float32)
    h3 = jnp.dot(x, w3_ref[0], preferred_element_type=jnp.float32)
    gate = g_ref[0, 0, :]
    o_ref[...] = (_silu(h1) * h3 * gate[:, None]).astype(jnp.bfloat16)


def _moe_stage1(block_e, x_pad, w1, w3, gate_r):
    grid_spec = pltpu.PrefetchScalarGridSpec(
        num_scalar_prefetch=1,
        grid=(FF // BN, N_MOE_BLOCKS),
        in_specs=[
            pl.BlockSpec((MOE_BM, D), lambda n, i, e: (i, 0)),
            pl.BlockSpec((1, D, BN), lambda n, i, e: (e[i], 0, n)),
            pl.BlockSpec((1, D, BN), lambda n, i, e: (e[i], 0, n)),
            pl.BlockSpec((1, 1, MOE_BM), lambda n, i, e: (i, 0, 0)),
        ],
        out_specs=pl.BlockSpec((MOE_BM, BN), lambda n, i, e: (i, n)),
    )
    return pl.pallas_call(
        _moe1_kernel,
        grid_spec=grid_spec,
        out_shape=jax.ShapeDtypeStruct((NPAD, FF), jnp.bfloat16),
    )(block_e, x_pad, w1, w3, gate_r)


def _moe2_kernel(e_ref, a_ref, w2_ref, o_ref):
    del e_ref
    o_ref[...] = jnp.dot(a_ref[...], w2_ref[0],
                         preferred_element_type=jnp.float32)


def _moe_stage2(block_e, act, w2):
    grid_spec = pltpu.PrefetchScalarGridSpec(
        num_scalar_prefetch=1,
        grid=(D // BN, N_MOE_BLOCKS),
        in_specs=[
            pl.BlockSpec((MOE_BM, FF), lambda n, i, e: (i, 0)),
            pl.BlockSpec((1, FF, BN), lambda n, i, e: (e[i], 0, n)),
        ],
        out_specs=pl.BlockSpec((MOE_BM, BN), lambda n, i, e: (i, n)),
    )
    return pl.pallas_call(
        _moe2_kernel,
        grid_spec=grid_spec,
        out_shape=jax.ShapeDtypeStruct((NPAD, D), jnp.float32),
    )(block_e, act, w2)


def _rope_tables():
    half = DH // 2
    inv = 1.0 / (10000.0 ** (np.arange(half, dtype=np.float32) / half))
    t = np.arange(S, dtype=np.float32)[:, None] * inv[None, :]
    return jnp.asarray(np.cos(t)), jnp.asarray(np.sin(t))


def _apply_rope(x, cos, sin):
    xh = x.reshape(S, H, DH)
    half = DH // 2
    x1, x2 = xh[..., :half], xh[..., half:]
    c = cos[:, None, :]
    s = sin[:, None, :]
    out = jnp.concatenate([x1 * c - x2 * s, x2 * c + x1 * s], axis=-1)
    return out.reshape(S, D)


def kernel(hidden_states, sequence_mask, load_balancing_loss, ln1_w, ln2_w,
           Wq, Wk, Wv, Wo, Wr, W1, W3, W2):
    x0 = hidden_states.reshape(S, D)

    wqkv = jnp.concatenate([Wq, Wk, Wv], axis=1)
    qkv = _ln_qkv(x0, ln1_w, wqkv)
    cos, sin = _rope_tables()
    bf = jnp.bfloat16
    q = _apply_rope(qkv[:, :D], cos, sin).reshape(S, H, DH).transpose(1, 0, 2)
    k = _apply_rope(qkv[:, D:2 * D], cos, sin).reshape(S, H, DH).transpose(1, 0, 2)
    v = qkv[:, 2 * D:].reshape(S, H, DH).transpose(1, 0, 2)
    attn = _attention(q.astype(bf), k.astype(bf), v.astype(bf))
    attn = attn.transpose(1, 0, 2).reshape(S, D)
    hidden = _proj_residual(attn.astype(bf), Wo.astype(bf), x0)

    wr_pad = jnp.zeros((D, 128), jnp.float32).at[:, :E].set(Wr)
    x2, logits_pad = _ln_router(hidden, ln2_w, wr_pad)
    logits = logits_pad[:, :E]
    probs = jax.nn.softmax(logits, axis=-1)
    topv, topi = jax.lax.top_k(probs, K)
    topvn = topv / jnp.sum(topv, axis=-1, keepdims=True)

    assign = jax.nn.one_hot(topi, E, dtype=jnp.float32).sum(axis=1)
    f = jnp.mean(assign, axis=0)
    P = jnp.mean(probs, axis=0)
    lbl = E * jnp.sum(f * P) / K

    T = S
    e_flat = topi.reshape(-1)
    gate_flat = topvn.reshape(-1)
    tok_flat = jnp.arange(T * K, dtype=jnp.int32) // K
    order = jnp.argsort(e_flat)
    sorted_e = e_flat[order]
    cnt = jnp.sum(jax.nn.one_hot(e_flat, E, dtype=jnp.int32), axis=0)
    off = jnp.concatenate([jnp.zeros((1,), jnp.int32), jnp.cumsum(cnt)[:-1]])
    padded_cnt = ((cnt + MOE_BM - 1) // MOE_BM) * MOE_BM
    pstart = jnp.concatenate(
        [jnp.zeros((1,), jnp.int32), jnp.cumsum(padded_cnt)[:-1]])
    rank = jnp.arange(T * K, dtype=jnp.int32) - off[sorted_e]
    dest = pstart[sorted_e] + rank
    tok_pad = jnp.zeros((NPAD,), jnp.int32).at[dest].set(tok_flat[order])
    gate_pad = jnp.zeros((NPAD,), jnp.float32).at[dest].set(gate_flat[order])
    inv = jnp.zeros((T * K,), jnp.int32).at[order].set(dest)
    block_e = (jnp.searchsorted(
        pstart, jnp.arange(N_MOE_BLOCKS, dtype=jnp.int32) * MOE_BM,
        side='right') - 1).astype(jnp.int32)

    x_pad = x2[tok_pad].astype(bf)
    gate_r = gate_pad.reshape(N_MOE_BLOCKS, 1, MOE_BM)
    act = _moe_stage1(block_e, x_pad, W1.astype(bf), W3.astype(bf), gate_r)
    y_pad = _moe_stage2(block_e, act, W2.astype(bf))

    inv2 = inv.reshape(T, K)
    out = y_pad[inv2[:, 0]] + y_pad[inv2[:, 1]] + hidden

    return (out.reshape(B, S, D), sequence_mask,
            load_balancing_loss + lbl.reshape(load_balancing_loss.shape))


# speedup vs baseline: 1.1911x; 1.1396x over previous
import functools
from typing import Any

import jax
import jax.numpy as jnp
import numpy as np
from jax.experimental import pallas as pl
from jax.experimental.pallas import tpu as pltpu

B, S, D, H, E, K, FF = 1, 2048, 1024, 16, 8, 2, 2048
DH = D // H
EPS = 1e-5

BM = 256
BN = 512
MOE_BM = 128
NPAD = (K * S // MOE_BM + E) * MOE_BM
N_MOE_BLOCKS = NPAD // MOE_BM


def _ln_qkv_kernel(x_ref, w_ref, wqkv_ref, o_ref):
    x = x_ref[...]
    var = jnp.mean(x * x, axis=1, keepdims=True)
    xn = (x * jax.lax.rsqrt(var + EPS) * w_ref[0, :]).astype(jnp.bfloat16)
    o_ref[...] = jnp.dot(xn, wqkv_ref[...], preferred_element_type=jnp.float32)


def _ln_qkv(x, ln_w, wqkv):
    return pl.pallas_call(
        _ln_qkv_kernel,
        grid=(S // BM, (3 * D) // BN),
        in_specs=[
            pl.BlockSpec((BM, D), lambda m, n: (m, 0)),
            pl.BlockSpec((1, D), lambda m, n: (0, 0)),
            pl.BlockSpec((D, BN), lambda m, n: (0, n)),
        ],
        out_specs=pl.BlockSpec((BM, BN), lambda m, n: (m, n)),
        out_shape=jax.ShapeDtypeStruct((S, 3 * D), jnp.float32),
    )(x, ln_w.reshape(1, D), wqkv.astype(jnp.bfloat16))


HG = 4


def _attn_kernel(q_ref, k_ref, v_ref, o_ref):
    qb = pl.program_id(1)
    row = qb * BM + jax.lax.broadcasted_iota(jnp.int32, (BM, S), 0)
    col = jax.lax.broadcasted_iota(jnp.int32, (BM, S), 1)
    causal = col <= row
    outs = []
    for h in range(HG):
        q = q_ref[:, h * DH:(h + 1) * DH]
        k = k_ref[:, h * DH:(h + 1) * DH]
        v = v_ref[:, h * DH:(h + 1) * DH]
        s = jax.lax.dot_general(q, k, (((1,), (1,)), ((), ())),
                                preferred_element_type=jnp.float32)
        s = s * (1.0 / np.sqrt(DH))
        s = jnp.where(causal, s, -1e9)
        m = jnp.max(s, axis=1, keepdims=True)
        p = jnp.exp(s - m)
        p = p / jnp.sum(p, axis=1, keepdims=True)
        outs.append(jnp.dot(p.astype(jnp.bfloat16), v,
                            preferred_element_type=jnp.float32))
    o_ref[...] = jnp.concatenate(outs, axis=1)


def _attention(qkv):
    return pl.pallas_call(
        _attn_kernel,
        grid=(H // HG, S // BM),
        in_specs=[
            pl.BlockSpec((BM, HG * DH), lambda hg, qb: (qb, hg)),
            pl.BlockSpec((S, HG * DH), lambda hg, qb: (0, H // HG + hg)),
            pl.BlockSpec((S, HG * DH), lambda hg, qb: (0, 2 * (H // HG) + hg)),
        ],
        out_specs=pl.BlockSpec((BM, HG * DH), lambda hg, qb: (qb, hg)),
        out_shape=jax.ShapeDtypeStruct((S, D), jnp.float32),
    )(qkv, qkv, qkv)


def _proj_res_kernel(a_ref, w_ref, r_ref, o_ref):
    o_ref[...] = jnp.dot(a_ref[...], w_ref[...],
                         preferred_element_type=jnp.float32) + r_ref[...]


def _silu(h):
    return h * jax.nn.sigmoid(h)


def _proj_residual(a, w, res):
    return pl.pallas_call(
        _proj_res_kernel,
        grid=(S // BM, D // BN),
        in_specs=[
            pl.BlockSpec((BM, D), lambda m, n: (m, 0)),
            pl.BlockSpec((D, BN), lambda m, n: (0, n)),
            pl.BlockSpec((BM, BN), lambda m, n: (m, n)),
        ],
        out_specs=pl.BlockSpec((BM, BN), lambda m, n: (m, n)),
        out_shape=jax.ShapeDtypeStruct((S, D), jnp.float32),
    )(a, w, res)


def _ln_router_kernel(x_ref, w_ref, wr_ref, xn_ref, lg_ref):
    x = x_ref[...]
    var = jnp.mean(x * x, axis=1, keepdims=True)
    xn = x * jax.lax.rsqrt(var + EPS) * w_ref[0, :]
    xn_ref[...] = xn
    lg_ref[...] = jnp.dot(xn, wr_ref[...], preferred_element_type=jnp.float32)


def _ln_router(x, ln_w, wr_pad):
    return pl.pallas_call(
        _ln_router_kernel,
        grid=(S // BM,),
        in_specs=[
            pl.BlockSpec((BM, D), lambda m: (m, 0)),
            pl.BlockSpec((1, D), lambda m: (0, 0)),
            pl.BlockSpec((D, 128), lambda m: (0, 0)),
        ],
        out_specs=[
            pl.BlockSpec((BM, D), lambda m: (m, 0)),
            pl.BlockSpec((BM, 128), lambda m: (m, 0)),
        ],
        out_shape=[
            jax.ShapeDtypeStruct((S, D), jnp.float32),
            jax.ShapeDtypeStruct((S, 128), jnp.float32),
        ],
    )(x, ln_w.reshape(1, D), wr_pad)


def _moe1_kernel(e_ref, x_ref, w1_ref, w3_ref, g_ref, o_ref):
    del e_ref
    x = x_ref[...]
    h1 = jnp.dot(x, w1_ref[0], preferred_element_type=jnp.float32)
    h3 = jnp.dot(x, w3_ref[0], preferred_element_type=jnp.float32)
    gate = g_ref[0, 0, :]
    o_ref[...] = (_silu(h1) * h3 * gate[:, None]).astype(jnp.bfloat16)


def _moe_stage1(block_e, x_pad, w1, w3, gate_r):
    grid_spec = pltpu.PrefetchScalarGridSpec(
        num_scalar_prefetch=1,
        grid=(FF // BN, N_MOE_BLOCKS),
        in_specs=[
            pl.BlockSpec((MOE_BM, D), lambda n, i, e: (i, 0)),
            pl.BlockSpec((1, D, BN), lambda n, i, e: (e[i], 0, n)),
            pl.BlockSpec((1, D, BN), lambda n, i, e: (e[i], 0, n)),
            pl.BlockSpec((1, 1, MOE_BM), lambda n, i, e: (i, 0, 0)),
        ],
        out_specs=pl.BlockSpec((MOE_BM, BN), lambda n, i, e: (i, n)),
    )
    return pl.pallas_call(
        _moe1_kernel,
        grid_spec=grid_spec,
        out_shape=jax.ShapeDtypeStruct((NPAD, FF), jnp.bfloat16),
    )(block_e, x_pad, w1, w3, gate_r)


def _moe2_kernel(e_ref, a_ref, w2_ref, o_ref):
    del e_ref
    o_ref[...] = jnp.dot(a_ref[...], w2_ref[0],
                         preferred_element_type=jnp.float32)


def _moe_stage2(block_e, act, w2):
    grid_spec = pltpu.PrefetchScalarGridSpec(
        num_scalar_prefetch=1,
        grid=(D // BN, N_MOE_BLOCKS),
        in_specs=[
            pl.BlockSpec((MOE_BM, FF), lambda n, i, e: (i, 0)),
            pl.BlockSpec((1, FF, BN), lambda n, i, e: (e[i], 0, n)),
        ],
        out_specs=pl.BlockSpec((MOE_BM, BN), lambda n, i, e: (i, n)),
    )
    return pl.pallas_call(
        _moe2_kernel,
        grid_spec=grid_spec,
        out_shape=jax.ShapeDtypeStruct((NPAD, D), jnp.float32),
    )(block_e, act, w2)


def _rope_tables():
    half = DH // 2
    inv = 1.0 / (10000.0 ** (np.arange(half, dtype=np.float32) / half))
    t = np.arange(S, dtype=np.float32)[:, None] * inv[None, :]
    return jnp.asarray(np.cos(t)), jnp.asarray(np.sin(t))


def _apply_rope(x, cos, sin):
    xh = x.reshape(S, H, DH)
    half = DH // 2
    x1, x2 = xh[..., :half], xh[..., half:]
    c = cos[:, None, :]
    s = sin[:, None, :]
    out = jnp.concatenate([x1 * c - x2 * s, x2 * c + x1 * s], axis=-1)
    return out.reshape(S, D)


def kernel(hidden_states, sequence_mask, load_balancing_loss, ln1_w, ln2_w,
           Wq, Wk, Wv, Wo, Wr, W1, W3, W2):
    x0 = hidden_states.reshape(S, D)

    wqkv = jnp.concatenate([Wq, Wk, Wv], axis=1)
    qkv = _ln_qkv(x0, ln1_w, wqkv)
    cos, sin = _rope_tables()
    bf = jnp.bfloat16
    qkv2 = jnp.concatenate([
        _apply_rope(qkv[:, :D], cos, sin),
        _apply_rope(qkv[:, D:2 * D], cos, sin),
        qkv[:, 2 * D:],
    ], axis=1).astype(bf)
    attn = _attention(qkv2)
    hidden = _proj_residual(attn.astype(bf), Wo.astype(bf), x0)

    wr_pad = jnp.zeros((D, 128), jnp.float32).at[:, :E].set(Wr)
    x2, logits_pad = _ln_router(hidden, ln2_w, wr_pad)
    logits = logits_pad[:, :E]
    probs = jax.nn.softmax(logits, axis=-1)
    topv, topi = jax.lax.top_k(probs, K)
    topvn = topv / jnp.sum(topv, axis=-1, keepdims=True)

    assign = jax.nn.one_hot(topi, E, dtype=jnp.float32).sum(axis=1)
    f = jnp.mean(assign, axis=0)
    P = jnp.mean(probs, axis=0)
    lbl = E * jnp.sum(f * P) / K

    T = S
    e_flat = topi.reshape(-1)
    gate_flat = topvn.reshape(-1)
    tok_flat = jnp.arange(T * K, dtype=jnp.int32) // K
    order = jnp.argsort(e_flat)
    sorted_e = e_flat[order]
    cnt = jnp.sum(jax.nn.one_hot(e_flat, E, dtype=jnp.int32), axis=0)
    off = jnp.concatenate([jnp.zeros((1,), jnp.int32), jnp.cumsum(cnt)[:-1]])
    padded_cnt = ((cnt + MOE_BM - 1) // MOE_BM) * MOE_BM
    pstart = jnp.concatenate(
        [jnp.zeros((1,), jnp.int32), jnp.cumsum(padded_cnt)[:-1]])
    rank = jnp.arange(T * K, dtype=jnp.int32) - off[sorted_e]
    dest = pstart[sorted_e] + rank
    tok_pad = jnp.zeros((NPAD,), jnp.int32).at[dest].set(tok_flat[order])
    gate_pad = jnp.zeros((NPAD,), jnp.float32).at[dest].set(gate_flat[order])
    inv = jnp.zeros((T * K,), jnp.int32).at[order].set(dest)
    block_e = (jnp.searchsorted(
        pstart, jnp.arange(N_MOE_BLOCKS, dtype=jnp.int32) * MOE_BM,
        side='right') - 1).astype(jnp.int32)

    x_pad = x2[tok_pad].astype(bf)
    gate_r = gate_pad.reshape(N_MOE_BLOCKS, 1, MOE_BM)
    act = _moe_stage1(block_e, x_pad, W1.astype(bf), W3.astype(bf), gate_r)
    y_pad = _moe_stage2(block_e, act, W2.astype(bf))

    inv2 = inv.reshape(T, K)
    out = y_pad[inv2[:, 0]] + y_pad[inv2[:, 1]] + hidden

    return (out.reshape(B, S, D), sequence_mask,
            load_balancing_loss + lbl.reshape(load_balancing_loss.shape))


# speedup vs baseline: 1.3396x; 1.1246x over previous
import functools
from typing import Any

import jax
import jax.numpy as jnp
import numpy as np
from jax.experimental import pallas as pl
from jax.experimental.pallas import tpu as pltpu

B, S, D, H, E, K, FF = 1, 2048, 1024, 16, 8, 2, 2048
DH = D // H
EPS = 1e-5

BM = 256
BN = 512
MOE_BM = 128
NPAD = (K * S // MOE_BM + E) * MOE_BM
N_MOE_BLOCKS = NPAD // MOE_BM


def _ln_qkv_kernel(x_ref, w_ref, wqkv_ref, c_ref, s_ref, o_ref):
    n = pl.program_id(1)
    x = x_ref[...]
    var = jnp.mean(x * x, axis=1, keepdims=True)
    xn = (x * jax.lax.rsqrt(var + EPS) * w_ref[0, :]).astype(jnp.bfloat16)
    h = jnp.dot(xn, wqkv_ref[...], preferred_element_type=jnp.float32)
    @pl.when(n < (2 * D) // BN)
    def _():
        c = c_ref[...]
        sn = s_ref[...]
        half = DH // 2
        pieces = []
        for g in range(BN // DH):
            x1 = h[:, g * DH:g * DH + half]
            x2 = h[:, g * DH + half:(g + 1) * DH]
            pieces.append(x1 * c - x2 * sn)
            pieces.append(x2 * c + x1 * sn)
        o_ref[...] = jnp.concatenate(pieces, axis=1).astype(jnp.bfloat16)

    @pl.when(n >= (2 * D) // BN)
    def _():
        o_ref[...] = h.astype(jnp.bfloat16)


def _ln_qkv(x, ln_w, wqkv, cos, sin):
    return pl.pallas_call(
        _ln_qkv_kernel,
        grid=(S // BM, (3 * D) // BN),
        in_specs=[
            pl.BlockSpec((BM, D), lambda m, n: (m, 0)),
            pl.BlockSpec((1, D), lambda m, n: (0, 0)),
            pl.BlockSpec((D, BN), lambda m, n: (0, n)),
            pl.BlockSpec((BM, DH // 2), lambda m, n: (m, 0)),
            pl.BlockSpec((BM, DH // 2), lambda m, n: (m, 0)),
        ],
        out_specs=pl.BlockSpec((BM, BN), lambda m, n: (m, n)),
        out_shape=jax.ShapeDtypeStruct((S, 3 * D), jnp.bfloat16),
    )(x, ln_w.reshape(1, D), wqkv.astype(jnp.bfloat16), cos, sin)


HG = 4


def _attn_kernel(q_ref, k_ref, v_ref, o_ref):
    qb = pl.program_id(1)
    row = qb * BM + jax.lax.broadcasted_iota(jnp.int32, (BM, S), 0)
    col = jax.lax.broadcasted_iota(jnp.int32, (BM, S), 1)
    causal = col <= row
    outs = []
    for h in range(HG):
        q = q_ref[:, h * DH:(h + 1) * DH]
        k = k_ref[:, h * DH:(h + 1) * DH]
        v = v_ref[:, h * DH:(h + 1) * DH]
        s = jax.lax.dot_general(q, k, (((1,), (1,)), ((), ())),
                                preferred_element_type=jnp.float32)
        s = s * (1.0 / np.sqrt(DH))
        s = jnp.where(causal, s, -1e9)
        m = jnp.max(s, axis=1, keepdims=True)
        p = jnp.exp(s - m)
        p = p / jnp.sum(p, axis=1, keepdims=True)
        outs.append(jnp.dot(p.astype(jnp.bfloat16), v,
                            preferred_element_type=jnp.float32))
    o_ref[...] = jnp.concatenate(outs, axis=1).astype(jnp.bfloat16)


def _attention(qkv):
    return pl.pallas_call(
        _attn_kernel,
        grid=(H // HG, S // BM),
        in_specs=[
            pl.BlockSpec((BM, HG * DH), lambda hg, qb: (qb, hg)),
            pl.BlockSpec((S, HG * DH), lambda hg, qb: (0, H // HG + hg)),
            pl.BlockSpec((S, HG * DH), lambda hg, qb: (0, 2 * (H // HG) + hg)),
        ],
        out_specs=pl.BlockSpec((BM, HG * DH), lambda hg, qb: (qb, hg)),
        out_shape=jax.ShapeDtypeStruct((S, D), jnp.bfloat16),
    )(qkv, qkv, qkv)


def _silu(h):
    return h * jax.nn.sigmoid(h)


def _wo_ln_router_kernel(a_ref, wo_ref, r_ref, w_ref, wr_ref,
                         hid_ref, xn_ref, lg_ref):
    hid = jnp.dot(a_ref[...], wo_ref[...],
                  preferred_element_type=jnp.float32) + r_ref[...]
    hid_ref[...] = hid
    var = jnp.mean(hid * hid, axis=1, keepdims=True)
    xn = hid * jax.lax.rsqrt(var + EPS) * w_ref[0, :]
    xn_ref[...] = xn
    lg_ref[...] = jnp.dot(xn, wr_ref[...], preferred_element_type=jnp.float32)


def _wo_ln_router(a, wo, res, ln_w, wr_pad):
    return pl.pallas_call(
        _wo_ln_router_kernel,
        grid=(S // BM,),
        in_specs=[
            pl.BlockSpec((BM, D), lambda m: (m, 0)),
            pl.BlockSpec((D, D), lambda m: (0, 0)),
            pl.BlockSpec((BM, D), lambda m: (m, 0)),
            pl.BlockSpec((1, D), lambda m: (0, 0)),
            pl.BlockSpec((D, 128), lambda m: (0, 0)),
        ],
        out_specs=[
            pl.BlockSpec((BM, D), lambda m: (m, 0)),
            pl.BlockSpec((BM, D), lambda m: (m, 0)),
            pl.BlockSpec((BM, 128), lambda m: (m, 0)),
        ],
        out_shape=[
            jax.ShapeDtypeStruct((S, D), jnp.float32),
            jax.ShapeDtypeStruct((S, D), jnp.float32),
            jax.ShapeDtypeStruct((S, 128), jnp.float32),
        ],
    )(a, wo, res, ln_w.reshape(1, D), wr_pad)


def _moe1_kernel(e_ref, x_ref, w1_ref, w3_ref, g_ref, o_ref):
    del e_ref
    x = x_ref[...]
    h1 = jnp.dot(x, w1_ref[0], preferred_element_type=jnp.float32)
    h3 = jnp.dot(x, w3_ref[0], preferred_element_type=jnp.float32)
    gate = g_ref[0, 0, :]
    o_ref[...] = (_silu(h1) * h3 * gate[:, None]).astype(jnp.bfloat16)


def _moe_stage1(block_e, x_pad, w1, w3, gate_r):
    grid_spec = pltpu.PrefetchScalarGridSpec(
        num_scalar_prefetch=1,
        grid=(FF // BN, N_MOE_BLOCKS),
        in_specs=[
            pl.BlockSpec((MOE_BM, D), lambda n, i, e: (i, 0)),
            pl.BlockSpec((1, D, BN), lambda n, i, e: (e[i], 0, n)),
            pl.BlockSpec((1, D, BN), lambda n, i, e: (e[i], 0, n)),
            pl.BlockSpec((1, 1, MOE_BM), lambda n, i, e: (i, 0, 0)),
        ],
        out_specs=pl.BlockSpec((MOE_BM, BN), lambda n, i, e: (i, n)),
    )
    return pl.pallas_call(
        _moe1_kernel,
        grid_spec=grid_spec,
        out_shape=jax.ShapeDtypeStruct((NPAD, FF), jnp.bfloat16),
    )(block_e, x_pad, w1, w3, gate_r)


def _moe2_kernel(e_ref, a_ref, w2_ref, o_ref):
    del e_ref
    o_ref[...] = jnp.dot(a_ref[...], w2_ref[0],
                         preferred_element_type=jnp.float32)


def _moe_stage2(block_e, act, w2):
    grid_spec = pltpu.PrefetchScalarGridSpec(
        num_scalar_prefetch=1,
        grid=(D // BN, N_MOE_BLOCKS),
        in_specs=[
            pl.BlockSpec((MOE_BM, FF), lambda n, i, e: (i, 0)),
            pl.BlockSpec((1, FF, BN), lambda n, i, e: (e[i], 0, n)),
        ],
        out_specs=pl.BlockSpec((MOE_BM, BN), lambda n, i, e: (i, n)),
    )
    return pl.pallas_call(
        _moe2_kernel,
        grid_spec=grid_spec,
        out_shape=jax.ShapeDtypeStruct((NPAD, D), jnp.float32),
    )(block_e, act, w2)


def _rope_tables():
    half = DH // 2
    inv = 1.0 / (10000.0 ** (np.arange(half, dtype=np.float32) / half))
    t = np.arange(S, dtype=np.float32)[:, None] * inv[None, :]
    return jnp.asarray(np.cos(t)), jnp.asarray(np.sin(t))


def _apply_rope(x, cos, sin):
    xh = x.reshape(S, H, DH)
    half = DH // 2
    x1, x2 = xh[..., :half], xh[..., half:]
    c = cos[:, None, :]
    s = sin[:, None, :]
    out = jnp.concatenate([x1 * c - x2 * s, x2 * c + x1 * s], axis=-1)
    return out.reshape(S, D)


def kernel(hidden_states, sequence_mask, load_balancing_loss, ln1_w, ln2_w,
           Wq, Wk, Wv, Wo, Wr, W1, W3, W2):
    x0 = hidden_states.reshape(S, D)
    bf = jnp.bfloat16

    wqkv = jnp.concatenate([Wq, Wk, Wv], axis=1)
    cos, sin = _rope_tables()
    qkv = _ln_qkv(x0, ln1_w, wqkv, cos, sin)
    attn = _attention(qkv)
    wr_pad = jnp.zeros((D, 128), jnp.float32).at[:, :E].set(Wr)
    hidden, x2, logits_pad = _wo_ln_router(
        attn, Wo.astype(bf), x0, ln2_w, wr_pad)
    logits = logits_pad[:, :E]
    probs = jax.nn.softmax(logits, axis=-1)
    topv, topi = jax.lax.top_k(probs, K)
    topvn = topv / jnp.sum(topv, axis=-1, keepdims=True)

    assign = jax.nn.one_hot(topi, E, dtype=jnp.float32).sum(axis=1)
    f = jnp.mean(assign, axis=0)
    P = jnp.mean(probs, axis=0)
    lbl = E * jnp.sum(f * P) / K

    T = S
    e_flat = topi.reshape(-1)
    gate_flat = topvn.reshape(-1)
    tok_flat = jnp.arange(T * K, dtype=jnp.int32) // K
    onehot = jax.nn.one_hot(e_flat, E, dtype=jnp.int32)
    cum = jnp.cumsum(onehot, axis=0)
    cnt = cum[-1]
    rank = jnp.take_along_axis(cum, e_flat[:, None], axis=1)[:, 0] - 1
    padded_cnt = ((cnt + MOE_BM - 1) // MOE_BM) * MOE_BM
    pstart = jnp.concatenate(
        [jnp.zeros((1,), jnp.int32), jnp.cumsum(padded_cnt)[:-1]])
    dest = pstart[e_flat] + rank
    tok_pad = jnp.zeros((NPAD,), jnp.int32).at[dest].set(tok_flat)
    gate_pad = jnp.zeros((NPAD,), jnp.float32).at[dest].set(gate_flat)
    block_e = (jnp.searchsorted(
        pstart, jnp.arange(N_MOE_BLOCKS, dtype=jnp.int32) * MOE_BM,
        side='right') - 1).astype(jnp.int32)

    x_pad = x2[tok_pad].astype(bf)
    gate_r = gate_pad.reshape(N_MOE_BLOCKS, 1, MOE_BM)
    act = _moe_stage1(block_e, x_pad, W1.astype(bf), W3.astype(bf), gate_r)
    y_pad = _moe_stage2(block_e, act, W2.astype(bf))

    inv2 = dest.reshape(T, K)
    out = y_pad[inv2[:, 0]] + y_pad[inv2[:, 1]] + hidden

    return (out.reshape(B, S, D), sequence_mask,
            load_balancing_loss + lbl.reshape(load_balancing_loss.shape))
